# baseline (speedup 1.0000x reference)
import functools

import jax
import jax.numpy as jnp
from jax import lax
from jax.experimental import pallas as pl
from jax.experimental.pallas import tpu as pltpu
from jax.experimental.pallas import tpu_sc as plsc

N = 100000
D = 128
G = 256
NW = 32
SC_N = 60000
TC_N = N - SC_N
RPW = SC_N // NW
C = 125
NCH = RPW // C
IDS_PAD = 1896
L = 16
TB = 2000
TG = TC_N // TB

assert NW * RPW == SC_N and NCH * C == RPW and TG * TB == TC_N

_mesh = plsc.VectorSubcoreMesh(core_axis_name="c", subcore_axis_name="s")


@functools.partial(
    pl.kernel,
    mesh=_mesh,
    compiler_params=pltpu.CompilerParams(needs_layout_passes=False),
    out_type=[
        jax.ShapeDtypeStruct((NW, G * D), jnp.float32),
        jax.ShapeDtypeStruct((NW, L), jnp.float32),
    ],
    scratch_types=[
        pltpu.VMEM(((C + 8) * D,), jnp.float32),
        pltpu.VMEM(((C + 8) * D,), jnp.float32),
        pltpu.VMEM((IDS_PAD,), jnp.int32),
        pltpu.VMEM((D,), jnp.float32),
        pltpu.VMEM((G * D,), jnp.float32),
        pltpu.VMEM((L,), jnp.float32),
        pltpu.SemaphoreType.DMA,
        pltpu.SemaphoreType.DMA,
    ],
)
def _sc_pool(x_hbm, ids_hbm, w_hbm, outp_hbm, outz_hbm,
             xb0, xb1, ids_v, w_v, acc, zbuf, sem0, sem1):
    wid = lax.axis_index("s") * 2 + lax.axis_index("c")
    row0 = wid * RPW

    pltpu.sync_copy(w_hbm, w_v)
    pltpu.sync_copy(ids_hbm.at[wid], ids_v)

    zeros = jnp.zeros((L,), jnp.float32)

    def _zero(t, carry):
        acc[pl.ds(t * L, L)] = zeros
        return carry

    lax.fori_loop(0, (G * D) // L, _zero, 0)

    wv = [w_v[pl.ds(k * L, L)] for k in range(8)]
    iota = lax.iota(jnp.int32, L)

    bufs = (xb0, xb1)
    sems = (sem0, sem1)

    def _issue(j, b):
        return pltpu.async_copy(
            x_hbm.at[pl.ds((row0 + j * C) * D, C * D)],
            bufs[b].at[pl.ds(0, C * D)], sems[b])

    rz = jnp.minimum(ids_v[pl.ds(0, L)][0], 0)

    def _process(buf, ids_base, z):
        @plsc.parallel_loop(0, C, unroll=5, carry=z)
        def zout(i, zc):
            base = i * D
            gv = ids_v[pl.ds(ids_base + i, L)]
            gi = jnp.broadcast_to(gv[0], (L,)) * D + iota
            xk = [buf[pl.ds(base + k * L, L)] for k in range(8)]
            d0 = xk[0] * wv[0] + xk[1] * wv[1]
            d1 = xk[2] * wv[2] + xk[3] * wv[3]
            d2 = xk[4] * wv[4] + xk[5] * wv[5]
            d3 = xk[6] * wv[6] + xk[7] * wv[7]
            dv = (d0 + d1) + (d2 + d3)
            s = jnp.sum(dv)
            e = jnp.exp(jnp.broadcast_to(s, (L,)))
            for k in range(8):
                xb = buf[pl.ds(base + rz + k * L, L)]
                plsc.addupdate_scatter(acc, [gi + (k * L)], e * xb)
            return zc + e

        return zout

    def _drain(b):
        pltpu.make_async_copy(
            x_hbm.at[pl.ds(row0 * D, C * D)],
            bufs[b].at[pl.ds(0, C * D)], sems[b]).wait()

    _issue(0, 0)
    _issue(1, 1)
    z = jnp.zeros((L,), jnp.float32)

    def _pair(jj, zc):
        j0 = jj * 2
        _drain(0)
        zc = _process(bufs[0], j0 * C, zc)
        _issue(j0 + 2, 0)
        _drain(1)
        zc = _process(bufs[1], (j0 + 1) * C, zc)

        @pl.when(jj < (NCH - 3) // 2)
        def _():
            _issue(j0 + 3, 1)

        return zc

    z = lax.fori_loop(0, (NCH - 1) // 2, _pair, z)
    _drain(0)
    z = _process(bufs[0], (NCH - 1) * C, z)

    zbuf[...] = z
    pltpu.sync_copy(acc, outp_hbm.at[wid])
    pltpu.sync_copy(zbuf, outz_hbm.at[wid])


def _tc_body(b_ref, x_ref, w_ref, o_ref, z_ref):
    i = pl.program_id(0)

    @pl.when(i == 0)
    def _():
        o_ref[...] = jnp.zeros_like(o_ref)
        z_ref[...] = jnp.zeros_like(z_ref)

    xb = x_ref[...]
    s = jnp.sum(xb * w_ref[...], axis=1, keepdims=True)
    e = jnp.exp(s)
    xw = xb * e
    bt = b_ref[0, 0, :]
    oh = (bt[None, :] ==
          lax.broadcasted_iota(jnp.int32, (G, TB), 0)).astype(jnp.float32)
    o_ref[...] += jnp.dot(oh, xw, preferred_element_type=jnp.float32)
    z_ref[...] += jnp.full((8, 128), jnp.sum(e), jnp.float32)


def _combine_body(p_ref, z_ref, ptc_ref, ztc_ref, o_ref):
    ztot = (jnp.sum(z_ref[...]) * (1.0 / L)
            + ztc_ref[0, 0])
    o_ref[...] = (jnp.sum(p_ref[...], axis=0) + ptc_ref[...]) * (1.0 / ztot)


def kernel(x, batch, W, b):
    del b
    ids32 = batch.astype(jnp.int32)
    xflat = x[:SC_N].reshape(-1)
    ids = ids32[:SC_N].reshape(NW, RPW)
    ids = jnp.pad(ids, ((0, 0), (0, IDS_PAD - RPW)))
    wflat = W.reshape(-1).astype(jnp.float32)

    partial, zp = _sc_pool(xflat, ids, wflat)

    p_tc, z_tc = pl.pallas_call(
        _tc_body,
        grid=(TG,),
        in_specs=[
            pl.BlockSpec((1, 1, TB), lambda i: (i, 0, 0)),
            pl.BlockSpec((TB, D), lambda i: (i, 0)),
            pl.BlockSpec((1, D), lambda i: (0, 0)),
        ],
        out_specs=[
            pl.BlockSpec((G, D), lambda i: (0, 0)),
            pl.BlockSpec((8, 128), lambda i: (0, 0)),
        ],
        out_shape=[
            jax.ShapeDtypeStruct((G, D), jnp.float32),
            jax.ShapeDtypeStruct((8, 128), jnp.float32),
        ],
    )(ids32[SC_N:].reshape(TG, 1, TB), x[SC_N:], W.astype(jnp.float32))

    out = pl.pallas_call(
        _combine_body,
        out_shape=jax.ShapeDtypeStruct((G, D), jnp.float32),
    )(partial.reshape(NW, G, D), zp, p_tc, z_tc)
    return out


# speedup vs baseline: 1.0934x; 1.0934x over previous
import functools

import jax
import jax.numpy as jnp
from jax import lax
from jax.experimental import pallas as pl
from jax.experimental.pallas import tpu as pltpu
from jax.experimental.pallas import tpu_sc as plsc

N = 100000
D = 128
G = 256
NW = 32
RPW = N // NW
C = 125
NCH = RPW // C
IDS_PAD = 3160
L = 16

assert NW * RPW == N and NCH * C == RPW

_mesh = plsc.VectorSubcoreMesh(core_axis_name="c", subcore_axis_name="s")


@functools.partial(
    pl.kernel,
    mesh=_mesh,
    compiler_params=pltpu.CompilerParams(needs_layout_passes=False),
    out_type=[
        jax.ShapeDtypeStruct((NW, G * D), jnp.float32),
        jax.ShapeDtypeStruct((NW, L), jnp.float32),
    ],
    scratch_types=[
        pltpu.VMEM(((C + 8) * D,), jnp.float32),
        pltpu.VMEM(((C + 8) * D,), jnp.float32),
        pltpu.VMEM((IDS_PAD,), jnp.int32),
        pltpu.VMEM((D,), jnp.float32),
        pltpu.VMEM((G * D,), jnp.float32),
        pltpu.VMEM((L,), jnp.float32),
        pltpu.SemaphoreType.DMA,
        pltpu.SemaphoreType.DMA,
    ],
)
def _sc_pool(x_hbm, ids_hbm, w_hbm, outp_hbm, outz_hbm,
             xb0, xb1, ids_v, w_v, acc, zbuf, sem0, sem1):
    wid = lax.axis_index("s") * 2 + lax.axis_index("c")
    row0 = wid * RPW

    pltpu.sync_copy(w_hbm, w_v)
    pltpu.sync_copy(ids_hbm.at[wid], ids_v)

    zeros = jnp.zeros((L,), jnp.float32)

    def _zero(t, carry):
        acc[pl.ds(t * L, L)] = zeros
        return carry

    lax.fori_loop(0, (G * D) // L, _zero, 0)

    wv = [w_v[pl.ds(k * L, L)] for k in range(8)]
    iota = lax.iota(jnp.int32, L)

    bufs = (xb0, xb1)
    sems = (sem0, sem1)

    def _issue(j, b):
        return pltpu.async_copy(
            x_hbm.at[pl.ds((row0 + j * C) * D, C * D)],
            bufs[b].at[pl.ds(0, C * D)], sems[b])

    rz = jnp.minimum(ids_v[pl.ds(0, L)][0], 0)

    def _process(buf, ids_base, z):
        R = 5

        @plsc.parallel_loop(0, C, step=R, carry=z)
        def zout(i, zc):
            gv = ids_v[pl.ds(ids_base + i, L)]
            gis = [jnp.broadcast_to(gv[r], (L,)) * D + iota for r in range(R)]
            bases = [(i + r) * D for r in range(R)]
            dvs = [None] * R
            for k in range(8):
                for r in range(R):
                    xk = buf[pl.ds(bases[r] + k * L, L)]
                    p = xk * wv[k]
                    dvs[r] = p if k == 0 else dvs[r] + p
            es = []
            for r in range(R):
                s = jnp.sum(dvs[r])
                es.append(jnp.exp(jnp.broadcast_to(s, (L,))))
                zc = zc + es[r]
            for k in range(8):
                vals = [es[r] * buf[pl.ds(bases[r] + rz + k * L, L)]
                        for r in range(R)]
                for r in range(R):
                    plsc.addupdate_scatter(acc, [gis[r] + (k * L)], vals[r])
            return zc

        return zout

    def _drain(b):
        pltpu.make_async_copy(
            x_hbm.at[pl.ds(row0 * D, C * D)],
            bufs[b].at[pl.ds(0, C * D)], sems[b]).wait()

    _issue(0, 0)
    _issue(1, 1)
    z = jnp.zeros((L,), jnp.float32)

    def _pair(jj, zc):
        j0 = jj * 2
        _drain(0)
        zc = _process(bufs[0], j0 * C, zc)
        _issue(j0 + 2, 0)
        _drain(1)
        zc = _process(bufs[1], (j0 + 1) * C, zc)

        @pl.when(jj < (NCH - 3) // 2)
        def _():
            _issue(j0 + 3, 1)

        return zc

    z = lax.fori_loop(0, (NCH - 1) // 2, _pair, z)
    _drain(0)
    z = _process(bufs[0], (NCH - 1) * C, z)

    zbuf[...] = z
    pltpu.sync_copy(acc, outp_hbm.at[wid])
    pltpu.sync_copy(zbuf, outz_hbm.at[wid])


def _combine_body(p_ref, z_ref, o_ref):
    ztot = jnp.sum(z_ref[...]) * (1.0 / L)
    o_ref[...] = jnp.sum(p_ref[...], axis=0) * (1.0 / ztot)


def kernel(x, batch, W, b):
    del b
    xflat = x.reshape(-1)
    ids = batch.astype(jnp.int32).reshape(NW, RPW)
    ids = jnp.pad(ids, ((0, 0), (0, IDS_PAD - RPW)))
    wflat = W.reshape(-1).astype(jnp.float32)

    partial, zp = _sc_pool(xflat, ids, wflat)

    out = pl.pallas_call(
        _combine_body,
        out_shape=jax.ShapeDtypeStruct((G, D), jnp.float32),
    )(partial.reshape(NW, G, D), zp)
    return out
